# baseline (speedup 1.0000x reference)
import jax
import jax.numpy as jnp
from jax.experimental import pallas as pl
from jax.experimental.pallas import tpu as pltpu

_SHIFT = 41.0
_GAMMA = 64.0


def _row_stats_kernel(sim_ref, label_ref, out_ref):
    s = sim_ref[0]
    lab = label_ref[0]

    pos = lab > 0.5
    neg = lab < 0.25

    tp = 0.8 - s
    ap = jnp.maximum(tp, 0.0)
    ep = jnp.exp(ap * tp * _GAMMA - _SHIFT)
    sum_p = jnp.sum(jnp.where(pos, ep, 0.0))

    tn = s - 0.2
    an = jnp.maximum(tn, 0.0)
    en = jnp.exp(an * tn * _GAMMA - _SHIFT)
    sum_n = jnp.sum(jnp.where(neg, en, 0.0))
    cnt_n = jnp.sum(jnp.where(neg, 1.0, 0.0))

    nm = jnp.where(neg, s, -1.0)

    def body(_, carry):
        nm, s_top, taken = carry
        m = jnp.max(nm)
        c = jnp.sum(jnp.where(nm == m, 1.0, 0.0))
        take = jnp.where(m > -0.5, jnp.minimum(c, 10.0 - taken), 0.0)
        tm = m - 0.2
        am = jnp.maximum(tm, 0.0)
        s_top = s_top + take * jnp.exp(am * tm * _GAMMA - _SHIFT)
        taken = taken + take
        nm = jnp.where(nm == m, -1.0, nm)
        return nm, s_top, taken

    _, sum_top, _ = jax.lax.fori_loop(0, 10, body, (nm, 0.0, 0.0))

    lane = jax.lax.broadcasted_iota(jnp.int32, (1, 128), 1)
    out = jnp.where(lane == 0, sum_p,
          jnp.where(lane == 1, sum_n,
          jnp.where(lane == 2, cnt_n,
          jnp.where(lane == 3, sum_top, 0.0))))
    out_ref[0] = out


def kernel(sim, label):
    b, n = sim.shape
    sim3 = sim.reshape(b, 1, n)
    label3 = label.reshape(b, 1, n)
    stats = pl.pallas_call(
        _row_stats_kernel,
        grid=(b,),
        in_specs=[
            pl.BlockSpec((1, 1, n), lambda i: (i, 0, 0)),
            pl.BlockSpec((1, 1, n), lambda i: (i, 0, 0)),
        ],
        out_specs=pl.BlockSpec((1, 1, 128), lambda i: (i, 0, 0)),
        out_shape=jax.ShapeDtypeStruct((b, 1, 128), jnp.float32),
    )(sim3, label3)

    sum_p = stats[:, 0, 0]
    sum_n = stats[:, 0, 1]
    cnt_n = stats[:, 0, 2]
    sum_top = stats[:, 0, 3]

    lse_p = jnp.where(sum_p > 0.0, jnp.log(sum_p) + _SHIFT, 0.0)
    lse_n = jnp.where(cnt_n > 20.5,
                      jnp.log(sum_top) + _SHIFT,
                      jnp.log(sum_n) + _SHIFT)
    loss = jnp.sum(jnp.logaddexp(lse_n + lse_p, 0.0)) / b
    return loss


# speedup vs baseline: 2.9839x; 2.9839x over previous
import jax
import jax.numpy as jnp
from jax import lax
from jax.experimental import pallas as pl
from jax.experimental.pallas import tpu as pltpu
from jax.experimental.pallas import tpu_sc as plsc

_SHIFT = 41.0
_GAMMA = 64.0
_NCHUNK = 10
_F = 10000
_G = 5
_NGROUP = _F // (16 * _G)
_LN2 = 0.6931471805599453


def _ln(x):
    bits = plsc.bitcast(x, jnp.int32)
    e = lax.shift_right_logical(bits, 23) - 127
    m = plsc.bitcast((bits & 0x007FFFFF) | 0x3F800000, jnp.float32)
    big = m > 1.4142135
    m = jnp.where(big, m * 0.5, m)
    ef = e.astype(jnp.float32) + jnp.where(big, 1.0, 0.0)
    t = (m - 1.0) / (m + 1.0)
    t2 = t * t
    ln_m = 2.0 * t * (1.0 + t2 * (1.0 / 3.0 + t2 * (0.2 + t2 * (1.0 / 7.0 + t2 / 9.0))))
    return ln_m + ef * _LN2


def _sc_body(sim_hbm, label_hbm, out_hbm, simbuf, labbuf, t_ref, res_ref):
    nc = 2
    wid = lax.axis_index("s") * nc + lax.axis_index("c")

    total = jnp.zeros((16,), jnp.float32)

    for r in range(2):
        row = wid * 2 + r

        t_ref[...] = jnp.full((16,), -1.0, jnp.float32)

        def chunk_body(c, carry):
            acc_p, acc_n, cnt_n = carry
            pltpu.sync_copy(sim_hbm.at[row, c], simbuf)
            pltpu.sync_copy(label_hbm.at[row, c], labbuf)

            def group_body(g, gcarry):
                acc_p, acc_n, cnt_n = gcarry
                base = g * (16 * _G)
                gmax = jnp.full((16,), -1.0, jnp.float32)
                for v in range(_G):
                    s = simbuf[pl.ds(base + v * 16, 16)]
                    labv = labbuf[pl.ds(base + v * 16, 16)]
                    pos = labv > 0.5
                    neg = labv < 0.25
                    tp = 0.8 - s
                    ep = jnp.exp(jnp.maximum(tp, 0.0) * tp * _GAMMA - _SHIFT)
                    acc_p = acc_p + jnp.where(pos, ep, 0.0)
                    tn = s - 0.2
                    en = jnp.exp(jnp.maximum(tn, 0.0) * tn * _GAMMA - _SHIFT)
                    acc_n = acc_n + jnp.where(neg, en, 0.0)
                    cnt_n = cnt_n + jnp.where(neg, 1.0, 0.0)
                    gmax = jnp.maximum(gmax, jnp.where(neg, s, -1.0))

                cur = t_ref[...]
                pred = jnp.max(gmax) > jnp.min(cur)

                @pl.when(pred)
                def _merge():
                    t = t_ref[...]
                    for v in range(_G):
                        s = simbuf[pl.ds(base + v * 16, 16)]
                        labv = labbuf[pl.ds(base + v * 16, 16)]
                        nm = jnp.where(labv < 0.25, s, -1.0)
                        snm = lax.sort(nm)
                        t = lax.sort(jnp.maximum(t, lax.rev(snm, (0,))))
                    t_ref[...] = t

                return acc_p, acc_n, cnt_n

            return lax.fori_loop(0, _NGROUP, group_body, (acc_p, acc_n, cnt_n))

        zero = jnp.zeros((16,), jnp.float32)
        acc_p, acc_n, cnt_n = lax.fori_loop(
            0, _NCHUNK, chunk_body, (zero, zero, zero))

        s_p = jnp.sum(acc_p)
        s_n = jnp.sum(acc_n)
        c_n = jnp.sum(cnt_n)

        t = t_ref[...]
        lane = lax.iota(jnp.int32, 16)
        tm = t - 0.2
        et = jnp.exp(jnp.maximum(tm, 0.0) * tm * _GAMMA - _SHIFT)
        s_top = jnp.sum(jnp.where(lane >= 6, et, 0.0))

        v_sp = jnp.full((16,), s_p)
        v_sn = jnp.full((16,), s_n)
        v_st = jnp.full((16,), s_top)
        v_cn = jnp.full((16,), c_n)

        lse_p = jnp.where(v_sp > 0.0, _ln(v_sp) + _SHIFT, 0.0)
        lse_n = jnp.where(v_cn > 20.5, _ln(v_st) + _SHIFT, _ln(v_sn) + _SHIFT)
        x = lse_n + lse_p
        softp = jnp.maximum(x, 0.0) + _ln(1.0 + jnp.exp(-jnp.abs(x)))
        total = total + jnp.where(v_cn > 0.5, softp, 0.0)

    res_ref[...] = total
    pltpu.sync_copy(res_ref, out_hbm.at[wid])


def kernel(sim, label):
    b, n = sim.shape
    sim3 = sim.reshape(b, _NCHUNK, _F)
    label3 = label.reshape(b, _NCHUNK, _F)
    k = pl.kernel(
        _sc_body,
        out_type=jax.ShapeDtypeStruct((32, 16), jnp.float32),
        mesh=plsc.VectorSubcoreMesh(
            core_axis_name="c", subcore_axis_name="s",
            num_cores=2, num_subcores=16),
        compiler_params=pltpu.CompilerParams(needs_layout_passes=False),
        scratch_types=[
            pltpu.VMEM((_F,), jnp.float32),
            pltpu.VMEM((_F,), jnp.float32),
            pltpu.VMEM((16,), jnp.float32),
            pltpu.VMEM((16,), jnp.float32),
        ],
    )
    out = k(sim3, label3)
    return jnp.sum(out[:, 0]) / b


# speedup vs baseline: 5.5449x; 1.8583x over previous
import jax
import jax.numpy as jnp
from jax import lax
from jax.experimental import pallas as pl
from jax.experimental.pallas import tpu as pltpu
from jax.experimental.pallas import tpu_sc as plsc

_SHIFT = 41.0
_GAMMA = 64.0
_NCHUNK = 10
_F = 10000
_G = 5
_GSZ = 16 * _G
_NSUB = 5
_GPS = _F // (_GSZ * _NSUB)
_LN2 = 0.6931471805599453


def _ln(x):
    bits = plsc.bitcast(x, jnp.int32)
    e = lax.shift_right_logical(bits, 23) - 127
    m = plsc.bitcast((bits & 0x007FFFFF) | 0x3F800000, jnp.float32)
    big = m > 1.4142135
    m = jnp.where(big, m * 0.5, m)
    ef = e.astype(jnp.float32) + jnp.where(big, 1.0, 0.0)
    t = (m - 1.0) / (m + 1.0)
    t2 = t * t
    ln_m = 2.0 * t * (1.0 + t2 * (1.0 / 3.0 + t2 * (0.2 + t2 * (1.0 / 7.0 + t2 / 9.0))))
    return ln_m + ef * _LN2


def _sc_body(sim_hbm, label_hbm, out_hbm,
             simbuf_a, labbuf_a, simbuf_b, labbuf_b,
             res_ref, pend_ref, cnt_ref, sem_a, sem_b):
    nc = 2
    wid = lax.axis_index("s") * nc + lax.axis_index("c")

    total = jnp.zeros((16,), jnp.float32)

    def process_chunk(simbuf, labbuf, carry):
        acc_p, acc_n, cnt_n, t16, tmin = carry

        def sub_body(sb, scarry):
            acc_p, acc_n, cnt_n, t16, tmin = scarry

            def group_body(gi, gcarry):
                acc_p, acc_n, cnt_n = gcarry
                base = (sb * _GPS + gi) * _GSZ
                gmax = jnp.full((16,), -1.0, jnp.float32)
                for v in range(_G):
                    s = simbuf[pl.ds(base + v * 16, 16)]
                    labv = labbuf[pl.ds(base + v * 16, 16)]
                    pos = labv > 0.5
                    neg = labv < 0.25
                    tp = 0.8 - s
                    ep = jnp.exp(jnp.maximum(tp, 0.0) * tp * _GAMMA - _SHIFT)
                    acc_p = acc_p + jnp.where(pos, ep, 0.0)
                    tn = s - 0.2
                    en = jnp.exp(jnp.maximum(tn, 0.0) * tn * _GAMMA - _SHIFT)
                    acc_n = acc_n + jnp.where(neg, en, 0.0)
                    cnt_n = cnt_n + jnp.where(neg, 1.0, 0.0)
                    gmax = jnp.maximum(gmax, jnp.where(neg, s, -1.0))

                @pl.when(jnp.max(gmax) > tmin)
                def _flag():
                    idx = cnt_ref[0]
                    pend_ref[idx] = gi
                    cnt_ref[0] = idx + 1

                return acc_p, acc_n, cnt_n

            acc_p, acc_n, cnt_n = lax.fori_loop(
                0, _GPS, group_body, (acc_p, acc_n, cnt_n))

            npend = cnt_ref[0]

            def drain_body(j, t):
                base = (sb * _GPS + pend_ref[j]) * _GSZ
                for v in range(_G):
                    s = simbuf[pl.ds(base + v * 16, 16)]
                    labv = labbuf[pl.ds(base + v * 16, 16)]
                    nm = jnp.where(labv < 0.25, s, -1.0)
                    snm = lax.sort(nm)
                    t = lax.sort(jnp.maximum(t, lax.rev(snm, (0,))))
                return t

            t16 = lax.fori_loop(0, npend, drain_body, t16)
            cnt_ref[0] = 0
            tmin = jnp.min(t16)
            return acc_p, acc_n, cnt_n, t16, tmin

        return lax.fori_loop(0, _NSUB, sub_body,
                             (acc_p, acc_n, cnt_n, t16, tmin))

    for r in range(2):
        row = wid * 2 + r
        cnt_ref[0] = 0

        pltpu.make_async_copy(sim_hbm.at[row, 0], simbuf_a, sem_a).start()
        pltpu.make_async_copy(label_hbm.at[row, 0], labbuf_a, sem_a).start()

        zero = jnp.zeros((16,), jnp.float32)
        carry0 = (zero, zero, zero, jnp.full((16,), -1.0, jnp.float32),
                  jnp.float32(-1.0))

        def pair_body(i, carry):
            c0 = 2 * i
            pltpu.make_async_copy(sim_hbm.at[row, c0 + 1],
                                  simbuf_b, sem_b).start()
            pltpu.make_async_copy(label_hbm.at[row, c0 + 1],
                                  labbuf_b, sem_b).start()
            pltpu.make_async_copy(sim_hbm.at[row, c0], simbuf_a, sem_a).wait()
            pltpu.make_async_copy(label_hbm.at[row, c0], labbuf_a, sem_a).wait()
            carry = process_chunk(simbuf_a, labbuf_a, carry)

            @pl.when(i < _NCHUNK // 2 - 1)
            def _next():
                pltpu.make_async_copy(sim_hbm.at[row, c0 + 2],
                                      simbuf_a, sem_a).start()
                pltpu.make_async_copy(label_hbm.at[row, c0 + 2],
                                      labbuf_a, sem_a).start()

            pltpu.make_async_copy(sim_hbm.at[row, c0 + 1],
                                  simbuf_b, sem_b).wait()
            pltpu.make_async_copy(label_hbm.at[row, c0 + 1],
                                  labbuf_b, sem_b).wait()
            return process_chunk(simbuf_b, labbuf_b, carry)

        acc_p, acc_n, cnt_n, t16, _ = lax.fori_loop(
            0, _NCHUNK // 2, pair_body, carry0)

        s_p = jnp.sum(acc_p)
        s_n = jnp.sum(acc_n)
        c_n = jnp.sum(cnt_n)

        lane = lax.iota(jnp.int32, 16)
        tm = t16 - 0.2
        et = jnp.exp(jnp.maximum(tm, 0.0) * tm * _GAMMA - _SHIFT)
        s_top = jnp.sum(jnp.where(lane >= 6, et, 0.0))

        v_sp = jnp.full((16,), s_p)
        v_sn = jnp.full((16,), s_n)
        v_st = jnp.full((16,), s_top)
        v_cn = jnp.full((16,), c_n)

        lse_p = jnp.where(v_sp > 0.0, _ln(v_sp) + _SHIFT, 0.0)
        lse_n = jnp.where(v_cn > 20.5, _ln(v_st) + _SHIFT, _ln(v_sn) + _SHIFT)
        x = lse_n + lse_p
        softp = jnp.maximum(x, 0.0) + _ln(1.0 + jnp.exp(-jnp.abs(x)))
        total = total + jnp.where(v_cn > 0.5, softp, 0.0)

    res_ref[...] = total
    pltpu.sync_copy(res_ref, out_hbm.at[wid])


def kernel(sim, label):
    b, n = sim.shape
    sim3 = sim.reshape(b, _NCHUNK, _F)
    label3 = label.reshape(b, _NCHUNK, _F)
    k = pl.kernel(
        _sc_body,
        out_type=jax.ShapeDtypeStruct((32, 16), jnp.float32),
        mesh=plsc.VectorSubcoreMesh(
            core_axis_name="c", subcore_axis_name="s",
            num_cores=2, num_subcores=16),
        compiler_params=pltpu.CompilerParams(needs_layout_passes=False),
        scratch_types=[
            pltpu.VMEM((_F,), jnp.float32),
            pltpu.VMEM((_F,), jnp.float32),
            pltpu.VMEM((_F,), jnp.float32),
            pltpu.VMEM((_F,), jnp.float32),
            pltpu.VMEM((16,), jnp.float32),
            pltpu.SMEM((32,), jnp.int32),
            pltpu.SMEM((1,), jnp.int32),
            pltpu.SemaphoreType.DMA,
            pltpu.SemaphoreType.DMA,
        ],
    )
    out = k(sim3, label3)
    return jnp.sum(out[:, 0]) / b
